# baseline (speedup 1.0000x reference)
import functools

import jax
import jax.numpy as jnp
from jax import lax
from jax.experimental import pallas as pl
from jax.experimental.pallas import tpu as pltpu
from jax.experimental.pallas import tpu_sc as plsc

B = 2048
K = 1024
D = 128

NC = 2
NS = 16
L = 16
NW = NC * NS
BPW = B // NW


def _tc_top2_body(w_ref, x_ref, i1_ref, i2_ref):
    w = w_ref[...]
    x = x_ref[...]
    wn = jnp.sum(w * w, axis=1, keepdims=True)
    xw = lax.dot_general(
        w, x,
        dimension_numbers=(((1,), (1,)), ((), ())),
        preferred_element_type=jnp.float32,
    )
    s = 0.5 * wn - xw
    io = lax.broadcasted_iota(jnp.int32, s.shape, 0)
    m1 = jnp.min(s, axis=0)
    i1 = jnp.min(jnp.where(s == m1[None, :], io, K), axis=0)
    s2 = jnp.where(io == i1[None, :], jnp.inf, s)
    m2 = jnp.min(s2, axis=0)
    i2 = jnp.min(jnp.where(s2 == m2[None, :], io, K), axis=0)
    i1_ref[...] = i1
    i2_ref[...] = i2


def _tc_top2(weights, inputs):
    return pl.pallas_call(
        _tc_top2_body,
        out_shape=[
            jax.ShapeDtypeStruct((B,), jnp.int32),
            jax.ShapeDtypeStruct((B,), jnp.int32),
        ],
    )(weights, inputs)


def _sc_refine_body(x_hbm, w_hbm, grid_hbm, i1_hbm, i2_hbm, out_hbm,
                    idx1_v, idx2_v, x_v, w1_v, w2_v, grid_v, out_v, sem):
    wid = lax.axis_index("s") * NC + lax.axis_index("c")
    base = wid * BPW
    a1 = pltpu.async_copy(i1_hbm.at[pl.ds(base, BPW)], idx1_v, sem)
    a2 = pltpu.async_copy(i2_hbm.at[pl.ds(base, BPW)], idx2_v, sem)
    a3 = pltpu.async_copy(grid_hbm, grid_v, sem)
    a4 = pltpu.async_copy(x_hbm.at[pl.ds(base, BPW)], x_v, sem)
    a1.wait()
    a2.wait()
    a3.wait()
    a4.wait()
    c1 = pltpu.async_copy(w_hbm.at[idx1_v], w1_v, sem)
    c2 = pltpu.async_copy(w_hbm.at[idx2_v], w2_v, sem)
    c1.wait()
    c2.wait()

    zero = jnp.zeros((L,), jnp.float32)
    zeros_i = jnp.zeros((L,), jnp.int32)
    ones_i = zeros_i + 1
    for g in range(BPW // L):
        rows = lax.iota(jnp.int32, L) + g * L
        i1v = idx1_v[pl.ds(g * L, L)]
        i2v = idx2_v[pl.ds(g * L, L)]

        def body(t, carry):
            a1, k1, a2, k2 = carry
            tv = jnp.full((L,), t, jnp.int32)
            xv = plsc.load_gather(x_v, [rows, tv])
            w1 = plsc.load_gather(w1_v, [rows, tv])
            w2 = plsc.load_gather(w2_v, [rows, tv])
            d1 = xv - w1
            d2 = xv - w2
            y1 = d1 * d1 - k1
            t1 = a1 + y1
            k1n = (t1 - a1) - y1
            y2 = d2 * d2 - k2
            t2 = a2 + y2
            k2n = (t2 - a2) - y2
            return (t1, k1n, t2, k2n)

        a1, _, a2, _ = lax.fori_loop(0, D, body, (zero, zero, zero, zero),
                                     unroll=8)
        take2 = (a2 < a1) | ((a2 == a1) & (i2v < i1v))
        win = jnp.where(take2, i2v, i1v)
        gx = plsc.load_gather(grid_v, [win, zeros_i])
        gy = plsc.load_gather(grid_v, [win, ones_i])
        plsc.store_scatter(out_v, [rows, zeros_i], gx)
        plsc.store_scatter(out_v, [rows, ones_i], gy)

    pltpu.sync_copy(out_v, out_hbm.at[pl.ds(base, BPW)])


@functools.cache
def _sc_refine():
    return pl.kernel(
        _sc_refine_body,
        mesh=plsc.VectorSubcoreMesh(
            core_axis_name="c", subcore_axis_name="s",
            num_cores=NC, num_subcores=NS,
        ),
        compiler_params=pltpu.CompilerParams(
            needs_layout_passes=False, use_tc_tiling_on_sc=False,
        ),
        out_type=jax.ShapeDtypeStruct((B, 2), jnp.int32),
        scratch_types=[
            pltpu.VMEM((BPW,), jnp.int32),
            pltpu.VMEM((BPW,), jnp.int32),
            pltpu.VMEM((BPW, D), jnp.float32),
            pltpu.VMEM((BPW, D), jnp.float32),
            pltpu.VMEM((BPW, D), jnp.float32),
            pltpu.VMEM((K, 2), jnp.int32),
            pltpu.VMEM((BPW, 2), jnp.int32),
            pltpu.SemaphoreType.DMA,
        ],
    )


def kernel(inputs, weights, grid):
    i1, i2 = _tc_top2(weights, inputs)
    return _sc_refine()(inputs, weights, grid, i1, i2)


# speedup vs baseline: 1.0581x; 1.0581x over previous
import functools

import jax
import jax.numpy as jnp
from jax import lax
from jax.experimental import pallas as pl
from jax.experimental.pallas import tpu as pltpu
from jax.experimental.pallas import tpu_sc as plsc

B = 2048
K = 1024
D = 128

NC = 2
NS = 16
L = 16
NW = NC * NS
BPW = B // NW


def _tc_top2_body(w_ref, x_ref, i1_ref, i2_ref):
    w = w_ref[...]
    x = x_ref[...]
    wn = jnp.sum(w * w, axis=1, keepdims=True)
    xw = lax.dot_general(
        w, x,
        dimension_numbers=(((1,), (1,)), ((), ())),
        preferred_element_type=jnp.float32,
        precision=lax.Precision.HIGHEST,
    )
    s = 0.5 * wn - xw
    io = lax.broadcasted_iota(jnp.int32, s.shape, 0)
    m1 = jnp.min(s, axis=0)
    i1 = jnp.min(jnp.where(s == m1[None, :], io, K), axis=0)
    s2 = jnp.where(io == i1[None, :], jnp.inf, s)
    m2 = jnp.min(s2, axis=0)
    i2 = jnp.min(jnp.where(s2 == m2[None, :], io, K), axis=0)
    i1_ref[...] = i1
    i2_ref[...] = i2


def _tc_top2(weights, inputs):
    return pl.pallas_call(
        _tc_top2_body,
        out_shape=[
            jax.ShapeDtypeStruct((B,), jnp.int32),
            jax.ShapeDtypeStruct((B,), jnp.int32),
        ],
    )(weights, inputs)


def _sc_gather_body(w_hbm, grid_hbm, i1_hbm, i2_hbm,
                    w1_hbm, w2_hbm, g1_hbm, g2_hbm,
                    idx1_v, idx2_v, w1_v, w2_v, grid_v, g1_v, g2_v, sem):
    wid = lax.axis_index("s") * NC + lax.axis_index("c")
    base = wid * BPW
    pltpu.sync_copy(i1_hbm.at[pl.ds(base, BPW)], idx1_v)
    pltpu.sync_copy(i2_hbm.at[pl.ds(base, BPW)], idx2_v)
    c1 = pltpu.async_copy(w_hbm.at[idx1_v], w1_v, sem)
    c2 = pltpu.async_copy(w_hbm.at[idx2_v], w2_v, sem)
    pltpu.sync_copy(grid_hbm, grid_v)

    zeros_i = jnp.zeros((L,), jnp.int32)
    ones_i = zeros_i + 1
    for g in range(BPW // L):
        rows = lax.iota(jnp.int32, L) + g * L
        i1v = idx1_v[pl.ds(g * L, L)]
        i2v = idx2_v[pl.ds(g * L, L)]
        gx1 = plsc.load_gather(grid_v, [i1v, zeros_i])
        gy1 = plsc.load_gather(grid_v, [i1v, ones_i])
        gx2 = plsc.load_gather(grid_v, [i2v, zeros_i])
        gy2 = plsc.load_gather(grid_v, [i2v, ones_i])
        plsc.store_scatter(g1_v, [rows, zeros_i], gx1)
        plsc.store_scatter(g1_v, [rows, ones_i], gy1)
        plsc.store_scatter(g2_v, [rows, zeros_i], gx2)
        plsc.store_scatter(g2_v, [rows, ones_i], gy2)

    c1.wait()
    c2.wait()
    pltpu.sync_copy(w1_v, w1_hbm.at[pl.ds(base, BPW)])
    pltpu.sync_copy(w2_v, w2_hbm.at[pl.ds(base, BPW)])
    pltpu.sync_copy(g1_v, g1_hbm.at[pl.ds(base, BPW)])
    pltpu.sync_copy(g2_v, g2_hbm.at[pl.ds(base, BPW)])


@functools.cache
def _sc_gather():
    return pl.kernel(
        _sc_gather_body,
        mesh=plsc.VectorSubcoreMesh(
            core_axis_name="c", subcore_axis_name="s",
            num_cores=NC, num_subcores=NS,
        ),
        compiler_params=pltpu.CompilerParams(
            needs_layout_passes=False, use_tc_tiling_on_sc=False,
        ),
        out_type=[
            jax.ShapeDtypeStruct((B, D), jnp.float32),
            jax.ShapeDtypeStruct((B, D), jnp.float32),
            jax.ShapeDtypeStruct((B, 2), jnp.int32),
            jax.ShapeDtypeStruct((B, 2), jnp.int32),
        ],
        scratch_types=[
            pltpu.VMEM((BPW,), jnp.int32),
            pltpu.VMEM((BPW,), jnp.int32),
            pltpu.VMEM((BPW, D), jnp.float32),
            pltpu.VMEM((BPW, D), jnp.float32),
            pltpu.VMEM((K, 2), jnp.int32),
            pltpu.VMEM((BPW, 2), jnp.int32),
            pltpu.VMEM((BPW, 2), jnp.int32),
            pltpu.SemaphoreType.DMA,
        ],
    )


def _tc_refine_body(x_ref, w1_ref, w2_ref, i1_ref, i2_ref, g1_ref, g2_ref,
                    out_ref):
    x = x_ref[...]
    e1 = x - w1_ref[...]
    e2 = x - w2_ref[...]
    d1 = jnp.sum(e1 * e1, axis=1)
    d2 = jnp.sum(e2 * e2, axis=1)
    i1 = i1_ref[...]
    i2 = i2_ref[...]
    take2 = (d2 < d1) | ((d2 == d1) & (i2 < i1))
    out_ref[...] = jnp.where(take2[:, None], g2_ref[...], g1_ref[...])


def _tc_refine(inputs, w1, w2, i1, i2, g1, g2):
    return pl.pallas_call(
        _tc_refine_body,
        out_shape=jax.ShapeDtypeStruct((B, 2), jnp.int32),
    )(inputs, w1, w2, i1, i2, g1, g2)


def kernel(inputs, weights, grid):
    i1, i2 = _tc_top2(weights, inputs)
    w1, w2, g1, g2 = _sc_gather()(weights, grid, i1, i2)
    return _tc_refine(inputs, w1, w2, i1, i2, g1, g2)
